# Initial kernel scaffold; baseline (speedup 1.0000x reference)
import jax
import jax.numpy as jnp
from jax.experimental import pallas as pl


def kernel(x_features, skip_features, W_trans, bn_t_g, bn_t_b, W1, bn1_g, bn1_b, W2, bn2_g, bn2_b, W_up, rb_trans_in, rb_trans_out, rb1_in, rb1_out, rb2_in, rb2_out, rb_up_in, rb_up_out):
    raise NotImplementedError("write your pallas kernel here")


# speedup vs baseline: 2.3480x; 2.3480x over previous
import functools

import jax
import jax.numpy as jnp
from jax import lax
from jax.experimental import pallas as pl
from jax.experimental.pallas import tpu as pltpu
from jax.experimental.pallas import tpu_sc as plsc

N = 50000
N_OUT = 100000
P = 12500
C_IN = 128
C_OUT = 64

NC = 2
NS = 16
LANES = 16
CHUNK = 128

BLK = 1000
EPS = 1e-5


def _cdiv(a, b):
    return (a + b - 1) // b


def _k1_body(x_ref, s_ref, w_ref, o_ref):
    x = x_ref[...] + s_ref[...]
    o_ref[...] = jnp.dot(x, w_ref[...], preferred_element_type=jnp.float32)


def _tc_mm1(x, skip, w_mat):
    kn = w_mat.shape[1]
    grid = N // BLK
    return pl.pallas_call(
        _k1_body,
        grid=(grid,),
        in_specs=[
            pl.BlockSpec((BLK, C_IN), lambda i: (i, 0)),
            pl.BlockSpec((BLK, C_IN), lambda i: (i, 0)),
            pl.BlockSpec((C_IN, kn), lambda i: (0, 0)),
        ],
        out_specs=pl.BlockSpec((BLK, kn), lambda i: (i, 0)),
        out_shape=jax.ShapeDtypeStruct((N, kn), jnp.float32),
    )(x, skip, w_mat)


def _k2_body(a_ref, s_ref, q_ref):
    a = a_ref[...]
    y = jnp.where(a >= 0, a, 0.01 * a)
    ps = jnp.sum(y, axis=1)
    pq = jnp.sum(y * y, axis=1)

    @pl.when(pl.program_id(0) == 0)
    def _():
        s_ref[...] = jnp.zeros_like(s_ref)
        q_ref[...] = jnp.zeros_like(q_ref)

    s_ref[...] += ps
    q_ref[...] += pq


def _tc_stats_leaky(a):
    grid = N // BLK
    return pl.pallas_call(
        _k2_body,
        grid=(grid,),
        in_specs=[pl.BlockSpec((2, BLK, 32), lambda i: (0, i, 0))],
        out_specs=[
            pl.BlockSpec((2, 32), lambda i: (0, 0)),
            pl.BlockSpec((2, 32), lambda i: (0, 0)),
        ],
        out_shape=[
            jax.ShapeDtypeStruct((2, 32), jnp.float32),
            jax.ShapeDtypeStruct((2, 32), jnp.float32),
        ],
    )(a)


def _k4_body(c1_ref, c2_ref, s1_ref, q1_ref, s2_ref, q2_ref):
    c1 = c1_ref[...]
    c2 = c2_ref[...]

    @pl.when(pl.program_id(0) == 0)
    def _():
        s1_ref[...] = jnp.zeros_like(s1_ref)
        q1_ref[...] = jnp.zeros_like(q1_ref)
        s2_ref[...] = jnp.zeros_like(s2_ref)
        q2_ref[...] = jnp.zeros_like(q2_ref)

    s1_ref[...] += jnp.sum(c1, axis=1)
    q1_ref[...] += jnp.sum(c1 * c1, axis=1)
    s2_ref[...] += jnp.sum(c2, axis=1)
    q2_ref[...] += jnp.sum(c2 * c2, axis=1)


def _tc_stats2(c1, c2):
    grid = N // BLK
    sspec = pl.BlockSpec((2, 32), lambda i: (0, 0))
    sshape = jax.ShapeDtypeStruct((2, 32), jnp.float32)
    return pl.pallas_call(
        _k4_body,
        grid=(grid,),
        in_specs=[
            pl.BlockSpec((2, BLK, 32), lambda i: (0, i, 0)),
            pl.BlockSpec((2, BLK, 32), lambda i: (0, i, 0)),
        ],
        out_specs=[sspec, sspec, sspec, sspec],
        out_shape=[sshape, sshape, sshape, sshape],
    )(c1, c2)


def _k3_body(a_ref, s_ref, q_ref, g_ref, b_ref, w_ref, o_ref):
    s = s_ref[...]
    q = q_ref[...]
    mu = s * (1.0 / N)
    var = q * (1.0 / N) - mu * mu
    sc = g_ref[...] * lax.rsqrt(var + EPS)
    off = b_ref[...] - mu * sc
    a = a_ref[...]
    y = jnp.where(a >= 0, a, 0.01 * a)
    y = y * sc[:, None, :] + off[:, None, :]
    cat = jnp.concatenate([y[0], y[1]], axis=-1)
    o_ref[...] = jnp.dot(cat, w_ref[...], preferred_element_type=jnp.float32)


def _tc_bn_mm(a, s, q, g2, b2, w_mat):
    kn = w_mat.shape[1]
    grid = N // BLK
    small = pl.BlockSpec((2, 32), lambda i: (0, 0))
    return pl.pallas_call(
        _k3_body,
        grid=(grid,),
        in_specs=[
            pl.BlockSpec((2, BLK, 32), lambda i: (0, i, 0)),
            small, small, small, small,
            pl.BlockSpec((C_OUT, kn), lambda i: (0, 0)),
        ],
        out_specs=pl.BlockSpec((BLK, kn), lambda i: (i, 0)),
        out_shape=jax.ShapeDtypeStruct((N, kn), jnp.float32),
    )(a, s, q, g2, b2, w_mat)


def _k5_body(c1_ref, c2_ref, s1_ref, q1_ref, s2_ref, q2_ref,
             g1_ref, b1_ref, g2_ref, b2_ref, w_ref, o_ref):
    def bn(x, s, q, g, b):
        mu = s * (1.0 / N)
        var = q * (1.0 / N) - mu * mu
        sc = g * lax.rsqrt(var + EPS)
        off = b - mu * sc
        return x * sc[:, None, :] + off[:, None, :]

    e = (bn(c1_ref[...], s1_ref[...], q1_ref[...], g1_ref[...], b1_ref[...])
         + bn(c2_ref[...], s2_ref[...], q2_ref[...], g2_ref[...], b2_ref[...]))
    cat = jnp.concatenate([e[0], e[1]], axis=-1)
    o_ref[...] = jnp.dot(cat, w_ref[...], preferred_element_type=jnp.float32)


def _tc_bn2_mm(c1, c2, s1, q1, s2, q2, g1h, b1h, g2h, b2h, w_mat):
    kn = w_mat.shape[1]
    grid = N // BLK
    small = pl.BlockSpec((2, 32), lambda i: (0, 0))
    return pl.pallas_call(
        _k5_body,
        grid=(grid,),
        in_specs=[
            pl.BlockSpec((2, BLK, 32), lambda i: (0, i, 0)),
            pl.BlockSpec((2, BLK, 32), lambda i: (0, i, 0)),
            small, small, small, small, small, small, small, small,
            pl.BlockSpec((C_OUT, kn), lambda i: (0, 0)),
        ],
        out_specs=pl.BlockSpec((BLK, kn), lambda i: (i, 0)),
        out_shape=jax.ShapeDtypeStruct((N, kn), jnp.float32),
    )(c1, c2, s1, q1, s2, q2, g1h, b1h, g2h, b2h, w_mat)


def _k6_body(a_ref, o_ref):
    a = a_ref[...]
    o_ref[...] = jnp.concatenate([a[0], a[1]], axis=-1)


def _tc_interleave(a, n_rows):
    grid = n_rows // BLK
    return pl.pallas_call(
        _k6_body,
        grid=(grid,),
        in_specs=[pl.BlockSpec((2, BLK, 32), lambda i: (0, i, 0))],
        out_specs=pl.BlockSpec((BLK, C_OUT), lambda i: (i, 0)),
        out_shape=jax.ShapeDtypeStruct((n_rows, C_OUT), jnp.float32),
    )(a)


ACC_ROWS = 50176
DUMMY = 50000
ZCHUNKS = ACC_ROWS // CHUNK
WCHUNKS = N // CHUNK
WTAIL = N - WCHUNKS * CHUNK


def _sc_zero_acc(s, zbuf, acc):
    for j in range(_cdiv(ZCHUNKS, NS)):
        ci = s + NS * j

        @pl.when(ci < ZCHUNKS)
        def _():
            pltpu.sync_copy(zbuf, acc.at[pl.ds(ci * CHUNK, CHUNK)])


def _sc_writeback(s, acc, out_hbm, out_base):
    for j in range(_cdiv(WCHUNKS, NS)):
        ci = s + NS * j

        @pl.when(ci < WCHUNKS)
        def _():
            pltpu.sync_copy(acc.at[pl.ds(ci * CHUNK, CHUNK)],
                            out_hbm.at[pl.ds(out_base + ci * CHUNK, CHUNK)])

    @pl.when(s == 0)
    def _():
        pltpu.sync_copy(acc.at[pl.ds(WCHUNKS * CHUNK, WTAIL)],
                        out_hbm.at[pl.ds(out_base + WCHUNKS * CHUNK, WTAIL)])


def _make_sc_conv(n_tab_rows, iters, n_pairs_pad):
    mesh = plsc.VectorSubcoreMesh(core_axis_name="c", subcore_axis_name="s")

    @functools.partial(
        pl.kernel,
        out_type=jax.ShapeDtypeStruct((2 * N, 32), jnp.float32),
        mesh=mesh,
        compiler_params=pltpu.CompilerParams(use_tc_tiling_on_sc=False),
        scratch_types=[
            pltpu.VMEM((CHUNK,), jnp.int32),
            pltpu.VMEM((CHUNK,), jnp.int32),
            pltpu.VMEM((CHUNK, 32), jnp.float32),
            pltpu.VMEM((CHUNK, 32), jnp.float32),
            pltpu.VMEM_SHARED((ACC_ROWS, 32), jnp.float32),
            pltpu.SemaphoreType.DMA,
        ],
    )
    def k(tab_hbm, gidx_hbm, sidx_hbm, zrows_hbm, out_hbm,
          idxg, idxs, rows, zbuf, acc, sem):
        c = lax.axis_index("c")
        s = lax.axis_index("s")
        pltpu.sync_copy(zrows_hbm, zbuf)
        _sc_zero_acc(s, zbuf, acc)
        plsc.subcore_barrier()

        span = iters * CHUNK
        base0 = s * span

        def body(i, _):
            base = base0 + i * CHUNK
            pltpu.sync_copy(gidx_hbm.at[pl.ds(base, CHUNK)], idxg)
            pltpu.sync_copy(sidx_hbm.at[pl.ds(base, CHUNK)], idxs)
            for j in range(CHUNK // LANES):
                sl = pl.ds(j * LANES, LANES)
                idxg[sl] = idxg[sl] + c
            pltpu.async_copy(tab_hbm.at[idxg], rows, sem).wait()
            pltpu.sync_copy(rows, acc.at[idxs], add=True)
            return 0

        lax.fori_loop(0, iters, body, 0)
        plsc.subcore_barrier()
        _sc_writeback(s, acc, out_hbm, c * N)

    return k


def _make_sc_conv2(n_tab_rows, iters):
    mesh = plsc.VectorSubcoreMesh(core_axis_name="c", subcore_axis_name="s")

    @functools.partial(
        pl.kernel,
        out_type=(jax.ShapeDtypeStruct((2 * N, 32), jnp.float32),
                  jax.ShapeDtypeStruct((2 * N, 32), jnp.float32)),
        mesh=mesh,
        compiler_params=pltpu.CompilerParams(use_tc_tiling_on_sc=False),
        scratch_types=[
            pltpu.VMEM((CHUNK,), jnp.int32),
            pltpu.VMEM((CHUNK,), jnp.int32),
            pltpu.VMEM((CHUNK, 32), jnp.float32),
            pltpu.VMEM((CHUNK, 32), jnp.float32),
            pltpu.VMEM_SHARED((ACC_ROWS, 32), jnp.float32),
            pltpu.SemaphoreType.DMA,
        ],
    )
    def k(tab_hbm, g1_hbm, s1_hbm, g2_hbm, s2_hbm, zrows_hbm,
          out1_hbm, out2_hbm, idxg, idxs, rows, zbuf, acc, sem):
        c = lax.axis_index("c")
        s = lax.axis_index("s")
        pltpu.sync_copy(zrows_hbm, zbuf)

        for gi, si, oi in ((g1_hbm, s1_hbm, out1_hbm),
                           (g2_hbm, s2_hbm, out2_hbm)):
            _sc_zero_acc(s, zbuf, acc)
            plsc.subcore_barrier()
            span = iters * CHUNK
            base0 = s * span

            def body(i, _, gi=gi, si=si):
                base = base0 + i * CHUNK
                pltpu.sync_copy(gi.at[pl.ds(base, CHUNK)], idxg)
                pltpu.sync_copy(si.at[pl.ds(base, CHUNK)], idxs)
                for j in range(CHUNK // LANES):
                    sl = pl.ds(j * LANES, LANES)
                    idxg[sl] = idxg[sl] + c
                pltpu.async_copy(tab_hbm.at[idxg], rows, sem).wait()
                pltpu.sync_copy(rows, acc.at[idxs], add=True)
                return 0

            lax.fori_loop(0, iters, body, 0)
            plsc.subcore_barrier()
            _sc_writeback(s, acc, oi, c * N)
            plsc.subcore_barrier()

    return k


def _make_sc_up(iters):
    mesh = plsc.VectorSubcoreMesh(core_axis_name="c", subcore_axis_name="s")

    @functools.partial(
        pl.kernel,
        out_type=jax.ShapeDtypeStruct((2 * N_OUT, 32), jnp.float32),
        mesh=mesh,
        compiler_params=pltpu.CompilerParams(use_tc_tiling_on_sc=False),
        scratch_types=[
            pltpu.VMEM((CHUNK,), jnp.int32),
            pltpu.VMEM((CHUNK,), jnp.int32),
            pltpu.VMEM((CHUNK, 32), jnp.float32),
            pltpu.VMEM((CHUNK, 32), jnp.float32),
            pltpu.VMEM_SHARED((ACC_ROWS, 32), jnp.float32),
            pltpu.SemaphoreType.DMA,
        ],
    )
    def k(tab_hbm, gidx_hbm, sidx_hbm, zrows_hbm, out_hbm,
          idxg, idxs, rows, zbuf, acc, sem):
        c = lax.axis_index("c")
        s = lax.axis_index("s")
        pltpu.sync_copy(zrows_hbm, zbuf)

        for p in range(2):
            _sc_zero_acc(s, zbuf, acc)
            plsc.subcore_barrier()
            span = iters * CHUNK
            base0 = s * span
            lo = p * N

            def body(i, _, lo=lo):
                base = base0 + i * CHUNK
                pltpu.sync_copy(gidx_hbm.at[pl.ds(base, CHUNK)], idxg)
                pltpu.sync_copy(sidx_hbm.at[pl.ds(base, CHUNK)], idxs)
                for j in range(CHUNK // LANES):
                    sl = pl.ds(j * LANES, LANES)
                    idxg[sl] = idxg[sl] + c
                    v = idxs[sl] - lo
                    ok = (v >= 0) & (v < N)
                    idxs[sl] = jnp.where(ok, v, DUMMY)
                pltpu.async_copy(tab_hbm.at[idxg], rows, sem).wait()
                pltpu.sync_copy(rows, acc.at[idxs], add=True)
                return 0

            lax.fori_loop(0, iters, body, 0)
            plsc.subcore_barrier()
            _sc_writeback(s, acc, out_hbm, c * N_OUT + p * N)
            plsc.subcore_barrier()

    return k


def _prep_gidx(rb_in, kk, koff, n_pairs_pad):
    k_ids = jnp.arange(rb_in.shape[0], dtype=jnp.int32)[:, None] + koff
    g = (rb_in.astype(jnp.int32) * kk + k_ids) * 2
    g = g.reshape(-1)
    return jnp.pad(g, (0, n_pairs_pad - g.shape[0]))


def _prep_sidx(rb_out, n_pairs_pad, fill):
    s = rb_out.astype(jnp.int32).reshape(-1)
    return jnp.pad(s, (0, n_pairs_pad - s.shape[0]), constant_values=fill)


def kernel(x_features, skip_features, W_trans, bn_t_g, bn_t_b, W1, bn1_g,
           bn1_b, W2, bn2_g, bn2_b, W_up, rb_trans_in, rb_trans_out, rb1_in,
           rb1_out, rb2_in, rb2_out, rb_up_in, rb_up_out):
    f32 = jnp.float32
    it27 = _cdiv(27 * P, NS * CHUNK)
    pad27 = NS * CHUNK * it27
    it9 = _cdiv(9 * P, NS * CHUNK)
    pad9 = NS * CHUNK * it9

    w_t = jnp.transpose(W_trans, (1, 0, 2)).reshape(C_IN, 27 * C_OUT)
    w_12 = jnp.transpose(jnp.concatenate([W1, W2], 0), (1, 0, 2))
    w_12 = w_12.reshape(C_OUT, 18 * C_OUT)
    w_up = jnp.transpose(W_up, (1, 0, 2)).reshape(C_OUT, 27 * C_OUT)

    gt = _prep_gidx(rb_trans_in, 27, 0, pad27)
    st = _prep_sidx(rb_trans_out, pad27, DUMMY)
    g1 = _prep_gidx(rb1_in, 18, 0, pad9)
    s1 = _prep_sidx(rb1_out, pad9, DUMMY)
    g2 = _prep_gidx(rb2_in, 18, 9, pad9)
    s2 = _prep_sidx(rb2_out, pad9, DUMMY)
    gu = _prep_gidx(rb_up_in, 27, 0, pad27)
    su = _prep_sidx(rb_up_out, pad27, jnp.int32(1 << 29))

    zrows = jnp.zeros((CHUNK, 32), f32)

    g2h_t = bn_t_g.reshape(2, 32).astype(f32)
    b2h_t = bn_t_b.reshape(2, 32).astype(f32)
    g2h_1 = bn1_g.reshape(2, 32).astype(f32)
    b2h_1 = bn1_b.reshape(2, 32).astype(f32)
    g2h_2 = bn2_g.reshape(2, 32).astype(f32)
    b2h_2 = bn2_b.reshape(2, 32).astype(f32)

    yt = _tc_mm1(x_features, skip_features, w_t)
    tab_t = yt.reshape(N * 54, 32)
    a_flat = _make_sc_conv(N * 54, it27, pad27)(tab_t, gt, st, zrows)
    a = a_flat.reshape(2, N, 32)

    s_t, q_t = _tc_stats_leaky(a)
    y12 = _tc_bn_mm(a, s_t, q_t, g2h_t, b2h_t, w_12)
    tab_12 = y12.reshape(N * 36, 32)

    c1_flat, c2_flat = _make_sc_conv2(N * 36, it9)(tab_12, g1, s1, g2, s2,
                                                   zrows)
    c1 = c1_flat.reshape(2, N, 32)
    c2 = c2_flat.reshape(2, N, 32)

    s1s, q1s, s2s, q2s = _tc_stats2(c1, c2)
    yup = _tc_bn2_mm(c1, c2, s1s, q1s, s2s, q2s,
                     g2h_1, b2h_1, g2h_2, b2h_2, w_up)
    tab_up = yup.reshape(N * 54, 32)

    o_flat = _make_sc_up(it27)(tab_up, gu, su, zrows)
    o = o_flat.reshape(2, N_OUT, 32)
    return _tc_interleave(o, N_OUT)
